# baseline (speedup 1.0000x reference)
import dataclasses
import functools

import jax
import jax.numpy as jnp
from jax import lax
from jax.experimental import pallas as pl
from jax.experimental.pallas import tpu as pltpu
from jax.experimental.pallas import tpu_sc as plsc

USERS = 4000
ITEMS = 6000
N = USERS + ITEMS
E = 320000
D = 128
H = 64
O = 32
B = 16384

NC = 2
NS = 16
NW = NC * NS
NPAD = N
EPT = E // NW
RPS = NPAD // NS
RSL = 640
PPT = B // NW
L = 16

f32 = jnp.float32
i32 = jnp.int32


@functools.lru_cache(maxsize=None)
def _sc_kernels():
    mesh = plsc.VectorSubcoreMesh(
        core_axis_name="c", subcore_axis_name="s",
        num_cores=NC, num_subcores=NS)
    cparams = pltpu.CompilerParams(use_tc_tiling_on_sc=False)
    if "needs_layout_passes" in pltpu.CompilerParams.__dataclass_fields__:
        cparams = dataclasses.replace(cparams, needs_layout_passes=False)

    def _row_base(sid):
        base = jnp.minimum((sid * RPS) // 8 * 8, NPAD - RSL)
        return pl.multiple_of(base, 8)

    def _fill(ref, n, value, width=None):
        vec = jnp.full((L,), value, dtype=ref.dtype)
        if width is None:
            def body(k, _):
                ref[pl.ds(k * L, L)] = vec
                return 0
            lax.fori_loop(0, n // L, body, 0)
        else:
            def body(r, _):
                for c in range(width // L):
                    ref[r, pl.ds(c * L, L)] = vec
                return 0
            lax.fori_loop(0, n, body, 0)

    @functools.partial(
        pl.kernel,
        out_type=jax.ShapeDtypeStruct((NC * NPAD,), f32),
        mesh=mesh,
        compiler_params=cparams,
        scratch_types=[
            pltpu.VMEM((EPT,), i32),
            pltpu.VMEM((EPT,), f32),
            pltpu.VMEM((RSL,), f32),
            pltpu.VMEM_SHARED((NPAD,), f32),
        ],
    )
    def _sc_deg(dst_hbm, out_hbm, idx_v, ones_v, buf_v, acc):
        cid = lax.axis_index("c")
        sid = lax.axis_index("s")
        wid = cid * NS + sid
        base = _row_base(sid)
        _fill(buf_v, RSL, 0.0)
        _fill(ones_v, EPT, 1.0)
        pltpu.sync_copy(buf_v, acc.at[pl.ds(base, RSL)])
        pltpu.sync_copy(dst_hbm.at[pl.ds(wid * EPT, EPT)], idx_v)
        plsc.subcore_barrier()
        pltpu.sync_copy(ones_v, acc.at[idx_v], add=True)
        plsc.subcore_barrier()
        pltpu.sync_copy(acc.at[pl.ds(base, RSL)], buf_v)
        pltpu.sync_copy(buf_v, out_hbm.at[pl.ds(cid * NPAD + base, RSL)])

    def _make_sc_agg(width, window):
        n_win = EPT // window
        assert n_win * window == EPT and (window % 8) == 0
        CH = min(window, RSL)
        chunks = [(off, min(CH, RSL - off)) for off in range(0, RSL, CH)]

        @functools.partial(
            pl.kernel,
            out_type=jax.ShapeDtypeStruct((NC, NPAD, width), f32),
            mesh=mesh,
            compiler_params=cparams,
            scratch_types=[
                pltpu.VMEM((EPT,), i32),
                pltpu.VMEM((window,), i32),
                pltpu.VMEM((window,), i32),
                pltpu.VMEM((window, width), f32),
                pltpu.VMEM((window, width), f32),
                pltpu.VMEM_SHARED((NPAD, width), f32),
                pltpu.SemaphoreType.DMA,
                pltpu.SemaphoreType.DMA,
                pltpu.SemaphoreType.DMA,
                pltpu.SemaphoreType.DMA,
            ],
        )
        def _agg(src_hbm, dst_hbm, y_hbm, out_hbm,
                 src_v, dst0_v, dst1_v, rows0_v, rows1_v, acc,
                 g0, g1, s0, s1):
            cid = lax.axis_index("c")
            sid = lax.axis_index("s")
            wid = cid * NS + sid
            base = _row_base(sid)
            dst_b = (dst0_v, dst1_v)
            rows_b = (rows0_v, rows1_v)
            gsem = (g0, g1)
            ssem = (s0, s1)

            _fill(rows0_v, CH, 0.0, width=width)
            for off, step in chunks:
                pltpu.sync_copy(rows0_v.at[pl.ds(0, step)],
                                acc.at[pl.ds(base + off, step)])
            pltpu.sync_copy(src_hbm.at[pl.ds(wid * EPT, EPT)], src_v)
            plsc.subcore_barrier()

            def _gather(w, b):
                pltpu.sync_copy(
                    dst_hbm.at[pl.ds(wid * EPT + w * window, window)], dst_b[b])
                return pltpu.async_copy(
                    y_hbm.at[src_v.at[pl.ds(w * window, window)]],
                    rows_b[b], gsem[b])

            gd = [None, None]
            sd = [None, None]
            gd[0] = _gather(0, 0)
            for w in range(n_win):
                cur = w & 1
                nxt = cur ^ 1
                gd[cur].wait()
                sd[cur] = pltpu.async_copy(
                    rows_b[cur], acc.at[dst_b[cur]], ssem[cur], add=True)
                if w + 1 < n_win:
                    if sd[nxt] is not None:
                        sd[nxt].wait()
                    gd[nxt] = _gather(w + 1, nxt)
            if n_win >= 2:
                sd[(n_win - 2) & 1].wait()
            sd[(n_win - 1) & 1].wait()
            plsc.subcore_barrier()
            for off, step in chunks:
                pltpu.sync_copy(acc.at[pl.ds(base + off, step)],
                                rows0_v.at[pl.ds(0, step)])
                pltpu.sync_copy(rows0_v.at[pl.ds(0, step)],
                                out_hbm.at[cid, pl.ds(base + off, step)])

        return _agg

    SCCH = 256

    @functools.partial(
        pl.kernel,
        out_type=jax.ShapeDtypeStruct((B,), f32),
        mesh=mesh,
        compiler_params=cparams,
        scratch_types=[
            pltpu.VMEM((PPT,), i32),
            pltpu.VMEM((PPT,), i32),
            pltpu.VMEM((SCCH, O), f32),
            pltpu.VMEM((SCCH, O), f32),
            pltpu.VMEM((PPT,), f32),
            pltpu.SemaphoreType.DMA,
        ],
    )
    def _sc_score(h2_hbm, uid_hbm, iid_hbm, out_hbm,
                  uid_v, iid_v, ur_v, ir_v, out_v, sem):
        wid = lax.axis_index("c") * NS + lax.axis_index("s")
        base = wid * PPT
        pltpu.sync_copy(uid_hbm.at[pl.ds(base, PPT)], uid_v)
        pltpu.sync_copy(iid_hbm.at[pl.ds(base, PPT)], iid_v)

        def _shift(k, _):
            iid_v[pl.ds(k * L, L)] = iid_v[pl.ds(k * L, L)] + USERS
            return 0

        lax.fori_loop(0, PPT // L, _shift, 0)
        lanes = lax.iota(i32, L)

        for ch in range(PPT // SCCH):
            pltpu.async_copy(
                h2_hbm.at[uid_v.at[pl.ds(ch * SCCH, SCCH)]], ur_v, sem).wait()
            pltpu.async_copy(
                h2_hbm.at[iid_v.at[pl.ds(ch * SCCH, SCCH)]], ir_v, sem).wait()

            def _dot(kk, _):
                rows = kk * L + lanes
                acc = jnp.zeros((L,), f32)
                for j in range(O):
                    jv = jnp.full((L,), j, dtype=i32)
                    u = plsc.load_gather(ur_v, [rows, jv])
                    v = plsc.load_gather(ir_v, [rows, jv])
                    acc = acc + u * v
                out_v[pl.ds(ch * SCCH + kk * L, L)] = acc
                return 0

            lax.fori_loop(0, SCCH // L, _dot, 0)
        pltpu.sync_copy(out_v, out_hbm.at[pl.ds(base, PPT)])

    return _sc_deg, _make_sc_agg(H, 400), _make_sc_agg(O, 1000), _sc_score


def _tc_b_body(degp_ref, feats_ref, w1_ref, y1_ref, dis_ref):
    deg = degp_ref[0] + degp_ref[1] + 1.0
    dis = lax.rsqrt(deg)
    xw = jnp.dot(feats_ref[...], w1_ref[...], preferred_element_type=f32)
    y1_ref[...] = dis * xw
    dis_ref[...] = dis


def _tc_d_body(aggp_ref, y1_ref, dis_ref, w2_ref, b1_ref, y2_ref):
    dis = dis_ref[...]
    h1 = dis * (aggp_ref[0] + aggp_ref[1] + y1_ref[...]) + b1_ref[...]
    y2_ref[...] = dis * jnp.dot(h1, w2_ref[...], preferred_element_type=f32)


def _tc_f_body(aggp_ref, y2_ref, dis_ref, b2_ref, h2_ref):
    h2_ref[...] = (dis_ref[...] * (aggp_ref[0] + aggp_ref[1] + y2_ref[...])
                   + b2_ref[...])


_tc_b = pl.pallas_call(
    _tc_b_body,
    out_shape=(jax.ShapeDtypeStruct((NPAD, H), f32),
               jax.ShapeDtypeStruct((NPAD, 1), f32)))

_tc_d = pl.pallas_call(
    _tc_d_body,
    out_shape=jax.ShapeDtypeStruct((NPAD, O), f32))

_tc_f = pl.pallas_call(
    _tc_f_body,
    out_shape=jax.ShapeDtypeStruct((NPAD, O), f32))


def kernel(x, edge_index, emb_user, emb_item, W1, b1, W2, b2):
    _sc_deg, _sc_agg64, _sc_agg32, _sc_score = _sc_kernels()
    feats = jnp.concatenate([emb_user, emb_item], axis=0)
    src = edge_index[0]
    dst = edge_index[1]
    uid = x[:, 0]
    iid = x[:, 1]

    degp = _sc_deg(dst).reshape(NC, NPAD, 1)
    y1, dis = _tc_b(degp, feats, W1)
    aggp1 = _sc_agg64(src, dst, y1)
    y2 = _tc_d(aggp1, y1, dis, W2, b1.reshape(1, H))
    aggp2 = _sc_agg32(src, dst, y2)
    h2 = _tc_f(aggp2, y2, dis, b2.reshape(1, O))
    out = _sc_score(h2, uid, iid)
    return out


# speedup vs baseline: 1.0573x; 1.0573x over previous
import dataclasses
import functools

import jax
import jax.numpy as jnp
from jax import lax
from jax.experimental import pallas as pl
from jax.experimental.pallas import tpu as pltpu
from jax.experimental.pallas import tpu_sc as plsc

USERS = 4000
ITEMS = 6000
N = USERS + ITEMS
E = 320000
D = 128
H = 64
O = 32
B = 16384

NC = 2
NS = 16
NW = NC * NS
NPAD = N
EPT = E // NW
RPS = NPAD // NS
RSL = 640
PPT = B // NW
L = 16

f32 = jnp.float32
i32 = jnp.int32


@functools.lru_cache(maxsize=None)
def _sc_kernels():
    mesh = plsc.VectorSubcoreMesh(
        core_axis_name="c", subcore_axis_name="s",
        num_cores=NC, num_subcores=NS)
    cparams = pltpu.CompilerParams(use_tc_tiling_on_sc=False)
    if "needs_layout_passes" in pltpu.CompilerParams.__dataclass_fields__:
        cparams = dataclasses.replace(cparams, needs_layout_passes=False)

    def _row_base(sid):
        base = jnp.minimum((sid * RPS) // 8 * 8, NPAD - RSL)
        return pl.multiple_of(base, 8)

    def _fill(ref, n, value, width=None):
        vec = jnp.full((L,), value, dtype=ref.dtype)
        if width is None:
            def body(k, _):
                ref[pl.ds(k * L, L)] = vec
                return 0
            lax.fori_loop(0, n // L, body, 0)
        else:
            def body(r, _):
                for c in range(width // L):
                    ref[r, pl.ds(c * L, L)] = vec
                return 0
            lax.fori_loop(0, n, body, 0)

    @functools.partial(
        pl.kernel,
        out_type=jax.ShapeDtypeStruct((NC * NPAD,), f32),
        mesh=mesh,
        compiler_params=cparams,
        scratch_types=[
            pltpu.VMEM((EPT,), i32),
            pltpu.VMEM((EPT,), f32),
            pltpu.VMEM((RSL,), f32),
            pltpu.VMEM_SHARED((NPAD,), f32),
        ],
    )
    def _sc_deg(dst_hbm, out_hbm, idx_v, ones_v, buf_v, acc):
        cid = lax.axis_index("c")
        sid = lax.axis_index("s")
        wid = cid * NS + sid
        base = _row_base(sid)
        _fill(buf_v, RSL, 0.0)
        _fill(ones_v, EPT, 1.0)
        pltpu.sync_copy(buf_v, acc.at[pl.ds(base, RSL)])
        pltpu.sync_copy(dst_hbm.at[pl.ds(wid * EPT, EPT)], idx_v)
        plsc.subcore_barrier()
        pltpu.sync_copy(ones_v, acc.at[idx_v], add=True)
        plsc.subcore_barrier()
        pltpu.sync_copy(acc.at[pl.ds(base, RSL)], buf_v)
        pltpu.sync_copy(buf_v, out_hbm.at[pl.ds(cid * NPAD + base, RSL)])

    def _make_sc_agg(width, window):
        n_win = EPT // window
        assert n_win * window == EPT and (window % 8) == 0
        CH = min(window, RSL)
        chunks = [(off, min(CH, RSL - off)) for off in range(0, RSL, CH)]

        @functools.partial(
            pl.kernel,
            out_type=jax.ShapeDtypeStruct((NC, NPAD, width), f32),
            mesh=mesh,
            compiler_params=cparams,
            scratch_types=[
                pltpu.VMEM((EPT,), i32),
                pltpu.VMEM((window,), i32),
                pltpu.VMEM((window,), i32),
                pltpu.VMEM((window, width), f32),
                pltpu.VMEM((window, width), f32),
                pltpu.VMEM_SHARED((NPAD, width), f32),
                pltpu.SemaphoreType.DMA,
                pltpu.SemaphoreType.DMA,
                pltpu.SemaphoreType.DMA,
                pltpu.SemaphoreType.DMA,
            ],
        )
        def _agg(src_hbm, dst_hbm, y_hbm, out_hbm,
                 src_v, dst0_v, dst1_v, rows0_v, rows1_v, acc,
                 g0, g1, s0, s1):
            cid = lax.axis_index("c")
            sid = lax.axis_index("s")
            wid = cid * NS + sid
            base = _row_base(sid)
            dst_b = (dst0_v, dst1_v)
            rows_b = (rows0_v, rows1_v)
            gsem = (g0, g1)
            ssem = (s0, s1)

            _fill(rows0_v, CH, 0.0, width=width)
            for off, step in chunks:
                pltpu.sync_copy(rows0_v.at[pl.ds(0, step)],
                                acc.at[pl.ds(base + off, step)])
            pltpu.sync_copy(src_hbm.at[pl.ds(wid * EPT, EPT)], src_v)
            plsc.subcore_barrier()

            def _gather(w, b):
                pltpu.sync_copy(
                    dst_hbm.at[pl.ds(wid * EPT + w * window, window)], dst_b[b])
                return pltpu.async_copy(
                    y_hbm.at[src_v.at[pl.ds(w * window, window)]],
                    rows_b[b], gsem[b])

            gd = [None, None]
            sd = [None, None]
            gd[0] = _gather(0, 0)
            for w in range(n_win):
                cur = w & 1
                nxt = cur ^ 1
                gd[cur].wait()
                sd[cur] = pltpu.async_copy(
                    rows_b[cur], acc.at[dst_b[cur]], ssem[cur], add=True)
                if w + 1 < n_win:
                    if sd[nxt] is not None:
                        sd[nxt].wait()
                    gd[nxt] = _gather(w + 1, nxt)
            if n_win >= 2:
                sd[(n_win - 2) & 1].wait()
            sd[(n_win - 1) & 1].wait()
            plsc.subcore_barrier()
            for off, step in chunks:
                pltpu.sync_copy(acc.at[pl.ds(base + off, step)],
                                rows0_v.at[pl.ds(0, step)])
                pltpu.sync_copy(rows0_v.at[pl.ds(0, step)],
                                out_hbm.at[cid, pl.ds(base + off, step)])

        return _agg

    SCCH = 256
    NROWS = USERS + USERS
    RPT8 = NROWS // NS
    RSL8 = 512

    @functools.partial(
        pl.kernel,
        out_type=jax.ShapeDtypeStruct((B,), f32),
        mesh=mesh,
        compiler_params=cparams,
        scratch_types=[
            pltpu.VMEM((PPT,), i32),
            pltpu.VMEM((PPT,), i32),
            pltpu.VMEM((SCCH, O), f32),
            pltpu.VMEM((SCCH, O), f32),
            pltpu.VMEM((PPT,), f32),
            pltpu.VMEM((RSL8, O), f32),
            pltpu.VMEM((RSL8, O), f32),
            pltpu.VMEM((RSL8, O), f32),
            pltpu.VMEM((RSL8,), f32),
            pltpu.VMEM((O,), f32),
            pltpu.VMEM_SHARED((NROWS, O), f32),
            pltpu.SemaphoreType.DMA,
        ],
    )
    def _sc_score(aggp_hbm, y2_hbm, dis_hbm, b2_hbm, uid_hbm, iid_hbm,
                  out_hbm, uid_v, iid_v, ur_v, ir_v, out_v,
                  p0_v, p1_v, y2_v, dis_v, b2_v, h2_sp, sem):
        cid = lax.axis_index("c")
        sid = lax.axis_index("s")
        wid = cid * NS + sid
        base = wid * PPT

        rb = pl.multiple_of(
            jnp.minimum((sid * RPT8) // 8 * 8, NROWS - RSL8), 8)
        rsl = pl.ds(rb, RSL8)
        pltpu.sync_copy(aggp_hbm.at[0, rsl], p0_v)
        pltpu.sync_copy(aggp_hbm.at[1, rsl], p1_v)
        pltpu.sync_copy(y2_hbm.at[rsl], y2_v)
        pltpu.sync_copy(dis_hbm.at[pl.ds(rb, RSL8)], dis_v)
        pltpu.sync_copy(b2_hbm.at[:], b2_v)
        b2c = [b2_v[pl.ds(c * L, L)] for c in range(O // L)]

        def _h2blk(g, _):
            dvec = dis_v[pl.ds(g * L, L)]
            for j in range(L):
                ds = dvec[j]
                r = g * L + j
                for c in range(O // L):
                    cc = pl.ds(c * L, L)
                    y2_v[r, cc] = (ds * (p0_v[r, cc] + p1_v[r, cc]
                                         + y2_v[r, cc]) + b2c[c])
            return 0

        lax.fori_loop(0, RSL8 // L, _h2blk, 0)
        pltpu.sync_copy(y2_v, h2_sp.at[rsl])
        plsc.subcore_barrier()

        pltpu.sync_copy(uid_hbm.at[pl.ds(base, PPT)], uid_v)
        pltpu.sync_copy(iid_hbm.at[pl.ds(base, PPT)], iid_v)

        def _shift(k, _):
            iid_v[pl.ds(k * L, L)] = iid_v[pl.ds(k * L, L)] + USERS
            return 0

        lax.fori_loop(0, PPT // L, _shift, 0)
        lanes = lax.iota(i32, L)

        for ch in range(PPT // SCCH):
            pltpu.async_copy(
                h2_sp.at[uid_v.at[pl.ds(ch * SCCH, SCCH)]], ur_v, sem).wait()
            pltpu.async_copy(
                h2_sp.at[iid_v.at[pl.ds(ch * SCCH, SCCH)]], ir_v, sem).wait()

            def _dot(kk, _):
                rows = kk * L + lanes
                acc = jnp.zeros((L,), f32)
                for j in range(O):
                    jv = jnp.full((L,), j, dtype=i32)
                    u = plsc.load_gather(ur_v, [rows, jv])
                    v = plsc.load_gather(ir_v, [rows, jv])
                    acc = acc + u * v
                out_v[pl.ds(ch * SCCH + kk * L, L)] = acc
                return 0

            lax.fori_loop(0, SCCH // L, _dot, 0)
        pltpu.sync_copy(out_v, out_hbm.at[pl.ds(base, PPT)])

    return _sc_deg, _make_sc_agg(H, 400), _make_sc_agg(O, 1000), _sc_score


def _tc_b_body(degp_ref, feats_ref, w1_ref, y1_ref, dis_ref):
    deg = degp_ref[0] + degp_ref[1] + 1.0
    dis = lax.rsqrt(deg)
    xw = jnp.dot(feats_ref[...], w1_ref[...], preferred_element_type=f32)
    y1_ref[...] = dis * xw
    dis_ref[...] = dis


def _tc_d_body(aggp_ref, y1_ref, dis_ref, w2_ref, b1_ref, y2_ref):
    dis = dis_ref[...]
    h1 = dis * (aggp_ref[0] + aggp_ref[1] + y1_ref[...]) + b1_ref[...]
    y2_ref[...] = dis * jnp.dot(h1, w2_ref[...], preferred_element_type=f32)


_tc_b = pl.pallas_call(
    _tc_b_body,
    out_shape=(jax.ShapeDtypeStruct((NPAD, H), f32),
               jax.ShapeDtypeStruct((NPAD, 1), f32)))

_tc_d = pl.pallas_call(
    _tc_d_body,
    out_shape=jax.ShapeDtypeStruct((NPAD, O), f32))


def kernel(x, edge_index, emb_user, emb_item, W1, b1, W2, b2):
    _sc_deg, _sc_agg64, _sc_agg32, _sc_score = _sc_kernels()
    feats = jnp.concatenate([emb_user, emb_item], axis=0)
    src = edge_index[0]
    dst = edge_index[1]
    uid = x[:, 0]
    iid = x[:, 1]

    degp = _sc_deg(dst).reshape(NC, NPAD, 1)
    y1, dis = _tc_b(degp, feats, W1)
    aggp1 = _sc_agg64(src, dst, y1)
    y2 = _tc_d(aggp1, y1, dis, W2, b1.reshape(1, H))
    aggp2 = _sc_agg32(src, dst, y2)
    out = _sc_score(aggp2, y2, dis.reshape(N), b2, uid, iid)
    return out


# speedup vs baseline: 1.0905x; 1.0314x over previous
import dataclasses
import functools

import jax
import jax.numpy as jnp
from jax import lax
from jax.experimental import pallas as pl
from jax.experimental.pallas import tpu as pltpu
from jax.experimental.pallas import tpu_sc as plsc

USERS = 4000
ITEMS = 6000
N = USERS + ITEMS
E = 320000
D = 128
H = 64
O = 32
B = 16384

NC = 2
NS = 16
NW = NC * NS
NPAD = N
EPT = E // NW
RPS = NPAD // NS
RSL = 640
PPT = B // NW
L = 16

f32 = jnp.float32
i32 = jnp.int32


@functools.lru_cache(maxsize=None)
def _sc_kernels():
    mesh = plsc.VectorSubcoreMesh(
        core_axis_name="c", subcore_axis_name="s",
        num_cores=NC, num_subcores=NS)
    cparams = pltpu.CompilerParams(use_tc_tiling_on_sc=False)
    if "needs_layout_passes" in pltpu.CompilerParams.__dataclass_fields__:
        cparams = dataclasses.replace(cparams, needs_layout_passes=False)

    def _row_base(sid):
        base = jnp.minimum((sid * RPS) // 8 * 8, NPAD - RSL)
        return pl.multiple_of(base, 8)

    def _fill(ref, n, value, width=None):
        vec = jnp.full((L,), value, dtype=ref.dtype)
        if width is None:
            def body(k, _):
                ref[pl.ds(k * L, L)] = vec
                return 0
            lax.fori_loop(0, n // L, body, 0)
        else:
            def body(r, _):
                for c in range(width // L):
                    ref[r, pl.ds(c * L, L)] = vec
                return 0
            lax.fori_loop(0, n, body, 0)

    @functools.partial(
        pl.kernel,
        out_type=jax.ShapeDtypeStruct((NC * NPAD,), f32),
        mesh=mesh,
        compiler_params=cparams,
        scratch_types=[
            pltpu.VMEM((EPT,), i32),
            pltpu.VMEM((EPT,), f32),
            pltpu.VMEM((RSL,), f32),
            pltpu.VMEM_SHARED((NPAD,), f32),
        ],
    )
    def _sc_deg(edge_hbm, out_hbm, idx_v, ones_v, buf_v, acc):
        cid = lax.axis_index("c")
        sid = lax.axis_index("s")
        wid = cid * NS + sid
        base = _row_base(sid)
        _fill(buf_v, RSL, 0.0)
        _fill(ones_v, EPT, 1.0)
        pltpu.sync_copy(buf_v, acc.at[pl.ds(base, RSL)])
        pltpu.sync_copy(edge_hbm.at[1, pl.ds(wid * EPT, EPT)], idx_v)
        plsc.subcore_barrier()
        pltpu.sync_copy(ones_v, acc.at[idx_v], add=True)
        plsc.subcore_barrier()
        pltpu.sync_copy(acc.at[pl.ds(base, RSL)], buf_v)
        pltpu.sync_copy(buf_v, out_hbm.at[pl.ds(cid * NPAD + base, RSL)])

    def _make_sc_agg(width, window):
        n_win = EPT // window
        assert n_win * window == EPT and (window % 8) == 0
        CH = min(window, RSL)
        chunks = [(off, min(CH, RSL - off)) for off in range(0, RSL, CH)]

        @functools.partial(
            pl.kernel,
            out_type=jax.ShapeDtypeStruct((NC, NPAD, width), f32),
            mesh=mesh,
            compiler_params=cparams,
            scratch_types=[
                pltpu.VMEM((EPT,), i32),
                pltpu.VMEM((window,), i32),
                pltpu.VMEM((window,), i32),
                pltpu.VMEM((window, width), f32),
                pltpu.VMEM((window, width), f32),
                pltpu.VMEM_SHARED((NPAD, width), f32),
                pltpu.SemaphoreType.DMA,
                pltpu.SemaphoreType.DMA,
                pltpu.SemaphoreType.DMA,
                pltpu.SemaphoreType.DMA,
            ],
        )
        def _agg(edge_hbm, y_hbm, out_hbm,
                 src_v, dst0_v, dst1_v, rows0_v, rows1_v, acc,
                 g0, g1, s0, s1):
            cid = lax.axis_index("c")
            sid = lax.axis_index("s")
            wid = cid * NS + sid
            base = _row_base(sid)
            dst_b = (dst0_v, dst1_v)
            rows_b = (rows0_v, rows1_v)
            gsem = (g0, g1)
            ssem = (s0, s1)

            _fill(rows0_v, CH, 0.0, width=width)
            for off, step in chunks:
                pltpu.sync_copy(rows0_v.at[pl.ds(0, step)],
                                acc.at[pl.ds(base + off, step)])
            pltpu.sync_copy(edge_hbm.at[0, pl.ds(wid * EPT, EPT)], src_v)
            plsc.subcore_barrier()

            def _gather(w, b):
                pltpu.sync_copy(
                    edge_hbm.at[1, pl.ds(wid * EPT + w * window, window)],
                    dst_b[b])
                return pltpu.async_copy(
                    y_hbm.at[src_v.at[pl.ds(w * window, window)]],
                    rows_b[b], gsem[b])

            gd = [None, None]
            sd = [None, None]
            gd[0] = _gather(0, 0)
            for w in range(n_win):
                cur = w & 1
                nxt = cur ^ 1
                gd[cur].wait()
                sd[cur] = pltpu.async_copy(
                    rows_b[cur], acc.at[dst_b[cur]], ssem[cur], add=True)
                if w + 1 < n_win:
                    if sd[nxt] is not None:
                        sd[nxt].wait()
                    gd[nxt] = _gather(w + 1, nxt)
            if n_win >= 2:
                sd[(n_win - 2) & 1].wait()
            sd[(n_win - 1) & 1].wait()
            plsc.subcore_barrier()
            for off, step in chunks:
                pltpu.sync_copy(acc.at[pl.ds(base + off, step)],
                                rows0_v.at[pl.ds(0, step)])
                pltpu.sync_copy(rows0_v.at[pl.ds(0, step)],
                                out_hbm.at[cid, pl.ds(base + off, step)])

        return _agg

    SCCH = 256
    NROWS = USERS + USERS
    RPT8 = NROWS // NS
    RSL8 = 512

    @functools.partial(
        pl.kernel,
        out_type=jax.ShapeDtypeStruct((B,), f32),
        mesh=mesh,
        compiler_params=cparams,
        scratch_types=[
            pltpu.VMEM((PPT,), i32),
            pltpu.VMEM((PPT,), i32),
            pltpu.VMEM((SCCH, O), f32),
            pltpu.VMEM((SCCH, O), f32),
            pltpu.VMEM((PPT,), f32),
            pltpu.VMEM((RSL8, O), f32),
            pltpu.VMEM((RSL8, O), f32),
            pltpu.VMEM((RSL8, O), f32),
            pltpu.VMEM((RSL8,), f32),
            pltpu.VMEM((O,), f32),
            pltpu.VMEM((PPT, 2), i32),
            pltpu.VMEM_SHARED((NROWS, O), f32),
            pltpu.SemaphoreType.DMA,
        ],
    )
    def _sc_score(aggp_hbm, y2_hbm, dis_hbm, b2_hbm, x_hbm,
                  out_hbm, uid_v, iid_v, ur_v, ir_v, out_v,
                  p0_v, p1_v, y2_v, dis_v, b2_v, ids_v, h2_sp, sem):
        cid = lax.axis_index("c")
        sid = lax.axis_index("s")
        wid = cid * NS + sid
        base = wid * PPT

        rb = pl.multiple_of(
            jnp.minimum((sid * RPT8) // 8 * 8, NROWS - RSL8), 8)
        rsl = pl.ds(rb, RSL8)
        pltpu.sync_copy(aggp_hbm.at[0, rsl], p0_v)
        pltpu.sync_copy(aggp_hbm.at[1, rsl], p1_v)
        pltpu.sync_copy(y2_hbm.at[rsl], y2_v)
        pltpu.sync_copy(dis_hbm.at[pl.ds(rb, RSL8)], dis_v)
        pltpu.sync_copy(b2_hbm.at[:], b2_v)
        b2c = [b2_v[pl.ds(c * L, L)] for c in range(O // L)]

        def _h2blk(g, _):
            dvec = dis_v[pl.ds(g * L, L)]
            for j in range(L):
                ds = dvec[j]
                r = g * L + j
                for c in range(O // L):
                    cc = pl.ds(c * L, L)
                    y2_v[r, cc] = (ds * (p0_v[r, cc] + p1_v[r, cc]
                                         + y2_v[r, cc]) + b2c[c])
            return 0

        lax.fori_loop(0, RSL8 // L, _h2blk, 0)
        pltpu.sync_copy(y2_v, h2_sp.at[rsl])
        plsc.subcore_barrier()

        pltpu.sync_copy(x_hbm.at[pl.ds(base, PPT)], ids_v)
        lanes = lax.iota(i32, L)
        col0 = jnp.zeros((L,), i32)
        col1 = jnp.ones((L,), i32)

        def _deint(k, _):
            rows = k * L + lanes
            uid_v[pl.ds(k * L, L)] = plsc.load_gather(ids_v, [rows, col0])
            iid_v[pl.ds(k * L, L)] = (
                plsc.load_gather(ids_v, [rows, col1]) + USERS)
            return 0

        lax.fori_loop(0, PPT // L, _deint, 0)

        for ch in range(PPT // SCCH):
            pltpu.async_copy(
                h2_sp.at[uid_v.at[pl.ds(ch * SCCH, SCCH)]], ur_v, sem).wait()
            pltpu.async_copy(
                h2_sp.at[iid_v.at[pl.ds(ch * SCCH, SCCH)]], ir_v, sem).wait()

            def _dot(kk, _):
                rows = kk * L + lanes
                acc = jnp.zeros((L,), f32)
                for j in range(O):
                    jv = jnp.full((L,), j, dtype=i32)
                    u = plsc.load_gather(ur_v, [rows, jv])
                    v = plsc.load_gather(ir_v, [rows, jv])
                    acc = acc + u * v
                out_v[pl.ds(ch * SCCH + kk * L, L)] = acc
                return 0

            lax.fori_loop(0, SCCH // L, _dot, 0)
        pltpu.sync_copy(out_v, out_hbm.at[pl.ds(base, PPT)])

    return _sc_deg, _make_sc_agg(H, 400), _make_sc_agg(O, 1000), _sc_score


def _tc_b_body(degp_ref, eu_ref, ei_ref, w1_ref, y1_ref, dis_ref):
    deg = degp_ref[0] + degp_ref[1] + 1.0
    dis = lax.rsqrt(deg)
    w1 = w1_ref[...]
    y1_ref[0:USERS, :] = dis[0:USERS] * jnp.dot(
        eu_ref[...], w1, preferred_element_type=f32)
    y1_ref[USERS:N, :] = dis[USERS:N] * jnp.dot(
        ei_ref[...], w1, preferred_element_type=f32)
    dis_ref[...] = dis


def _tc_d_body(aggp_ref, y1_ref, dis_ref, w2_ref, b1_ref, y2_ref):
    dis = dis_ref[...]
    h1 = dis * (aggp_ref[0] + aggp_ref[1] + y1_ref[...]) + b1_ref[...]
    y2_ref[...] = dis * jnp.dot(h1, w2_ref[...], preferred_element_type=f32)


_tc_b = pl.pallas_call(
    _tc_b_body,
    out_shape=(jax.ShapeDtypeStruct((NPAD, H), f32),
               jax.ShapeDtypeStruct((NPAD, 1), f32)))

_tc_d = pl.pallas_call(
    _tc_d_body,
    out_shape=jax.ShapeDtypeStruct((NPAD, O), f32))


def kernel(x, edge_index, emb_user, emb_item, W1, b1, W2, b2):
    _sc_deg, _sc_agg64, _sc_agg32, _sc_score = _sc_kernels()
    degp = _sc_deg(edge_index).reshape(NC, NPAD, 1)
    y1, dis = _tc_b(degp, emb_user, emb_item, W1)
    aggp1 = _sc_agg64(edge_index, y1)
    y2 = _tc_d(aggp1, y1, dis, W2, b1.reshape(1, H))
    aggp2 = _sc_agg32(edge_index, y2)
    out = _sc_score(aggp2, y2, dis.reshape(N), b2, x)
    return out
